# baseline (speedup 1.0000x reference)
import functools

import jax
import jax.numpy as jnp
from jax import lax
from jax.experimental import pallas as pl
from jax.experimental.pallas import tpu as pltpu
from jax.experimental.pallas import tpu_sc as plsc

B, T, D = 4, 8192, 512
BOT = 64
N = B * T

_BT = 2048
_NBLK = N // _BT
_CH = 64
CHUNK_BLOCKS = (2, 2, 4, 4, 4)
assert sum(CHUNK_BLOCKS) == _NBLK


@functools.cache
def _make_sc_gather(rows):
    info = plsc.get_sparse_core_info()
    nc, ns = info.num_cores, info.num_subcores
    nw = nc * ns
    rpw = rows // nw
    stages = rpw // _CH
    mesh = plsc.VectorSubcoreMesh(core_axis_name="c", subcore_axis_name="s")

    @functools.partial(
        pl.kernel,
        mesh=mesh,
        out_type=jax.ShapeDtypeStruct((rows, D), jnp.float32),
        scratch_types=[
            pltpu.VMEM((rpw,), jnp.int32),
            pltpu.VMEM((_CH, D), jnp.float32),
            pltpu.VMEM((_CH, D), jnp.float32),
            pltpu.SemaphoreType.DMA,
            pltpu.SemaphoreType.DMA,
            pltpu.SemaphoreType.DMA,
            pltpu.SemaphoreType.DMA,
        ],
    )
    def sc_gather(x_hbm, idx_hbm, out_hbm, idx_v, buf0, buf1,
                  gs0, gs1, ws0, ws1):
        wid = lax.axis_index("s") * nc + lax.axis_index("c")
        pltpu.sync_copy(idx_hbm.at[pl.ds(wid * rpw, rpw)], idx_v)
        bufs = (buf0, buf1)
        gsems = (gs0, gs1)
        wsems = (ws0, ws1)
        gathers = [None, None]
        writes = [None, None]
        for j in range(stages):
            p = j % 2
            if j >= 2:
                writes[p].wait()
            gathers[p] = pltpu.make_async_copy(
                x_hbm.at[idx_v.at[pl.ds(j * _CH, _CH)]], bufs[p], gsems[p])
            gathers[p].start()
            if j >= 1:
                q = 1 - p
                gathers[q].wait()
                writes[q] = pltpu.make_async_copy(
                    bufs[q], out_hbm.at[pl.ds((wid * stages + j - 1) * _CH,
                                              _CH)], wsems[q])
                writes[q].start()
        pl_ = (stages - 1) % 2
        gathers[pl_].wait()
        writes[pl_] = pltpu.make_async_copy(
            bufs[pl_], out_hbm.at[pl.ds((wid * stages + stages - 1) * _CH,
                                        _CH)], wsems[pl_])
        writes[pl_].start()
        writes[pl_].wait()
        if stages >= 2:
            writes[1 - pl_].wait()

    return sc_gather


def _tc_body(blk0, x_ref, agg_ref, idxf_ref, dw_ref, db_ref, uw_ref, ub_ref,
             gw_ref, gb_ref, *rest):
    out_ref = rest[-1]
    xb = x_ref[...]
    ab = agg_ref[...]
    h = jnp.maximum(
        jnp.dot(ab, dw_ref[...], preferred_element_type=jnp.float32)
        + db_ref[...], 0.0)
    z = jnp.dot(h, uw_ref[...], preferred_element_type=jnp.float32) + ub_ref[...]
    garg = (jnp.dot(xb, gw_ref[:D, :], preferred_element_type=jnp.float32)
            + jnp.dot(ab, gw_ref[D:, :], preferred_element_type=jnp.float32)
            + gb_ref[...])
    g = jax.nn.sigmoid(garg)
    r0 = (blk0 + pl.program_id(0)) * _BT
    rows = r0 + lax.broadcasted_iota(jnp.int32, (_BT, 1), 0)
    t = rows % T
    nbr = jnp.reshape(idxf_ref[...], (_BT, 1)) - (rows // T) * T
    mask = ((t >= 1) & (nbr != 0)).astype(jnp.float32)
    out_ref[...] = xb + (g * z) * mask


def _tc_dense_chunk(blk0, nblk, x2, agg_c, idxf2, dw, db, uw, ub, gw, gb,
                    acc=None):
    in_specs = [
        pl.BlockSpec((_BT, D), lambda i: (blk0 + i, 0)),
        pl.BlockSpec((_BT, D), lambda i: (i, 0)),
        pl.BlockSpec((1, _BT), lambda i: (0, blk0 + i)),
        pl.BlockSpec((D, BOT), lambda i: (0, 0)),
        pl.BlockSpec((1, BOT), lambda i: (0, 0)),
        pl.BlockSpec((BOT, D), lambda i: (0, 0)),
        pl.BlockSpec((1, D), lambda i: (0, 0)),
        pl.BlockSpec((2 * D, D), lambda i: (0, 0)),
        pl.BlockSpec((1, D), lambda i: (0, 0)),
    ]
    args = [x2, agg_c, idxf2, dw, db, uw, ub, gw, gb]
    aliases = {}
    if acc is not None:
        in_specs.append(pl.BlockSpec((8, 128), lambda i: (0, 0)))
        args.append(acc)
        aliases = {9: 0}
    return pl.pallas_call(
        functools.partial(_tc_body, blk0),
        grid=(nblk,),
        in_specs=in_specs,
        out_specs=pl.BlockSpec((_BT, D), lambda i: (blk0 + i, 0)),
        out_shape=jax.ShapeDtypeStruct((N, D), jnp.float32),
        input_output_aliases=aliases,
    )(*args)


def kernel(x, neighbor_idx, down_w, down_b, up_w, up_b, gate_w, gate_b):
    idx = neighbor_idx[:, 0]
    idxf = ((jnp.arange(B, dtype=jnp.int32) * T)[:, None]
            + idx[None, :]).reshape(N)
    x2 = x.reshape(N, D)
    offs = [0]
    for nb in CHUNK_BLOCKS:
        offs.append(offs[-1] + nb)
    aggs = [
        _make_sc_gather(nb * _BT)(
            x2,
            lax.slice(idxf, (offs[c] * _BT,), (offs[c + 1] * _BT,)))
        for c, nb in enumerate(CHUNK_BLOCKS)
    ]
    idxf2 = idxf.reshape(1, N)
    db2, ub2, gb2 = (down_b.reshape(1, BOT), up_b.reshape(1, D),
                     gate_b.reshape(1, D))
    acc = None
    for c, nb in enumerate(CHUNK_BLOCKS):
        acc = _tc_dense_chunk(offs[c], nb, x2, aggs[c], idxf2, down_w, db2,
                              up_w, ub2, gate_w, gb2, acc)
    return acc.reshape(B, T, D)


# speedup vs baseline: 1.0286x; 1.0286x over previous
import functools

import jax
import jax.numpy as jnp
from jax import lax
from jax.experimental import pallas as pl
from jax.experimental.pallas import tpu as pltpu
from jax.experimental.pallas import tpu_sc as plsc

B, T, D = 4, 8192, 512
BOT = 64
N = B * T

_BT = 1024
_NBLK = N // _BT
_CH = 64
CHUNK_BLOCKS = (4, 4, 8, 8, 8)
assert sum(CHUNK_BLOCKS) == _NBLK


@functools.cache
def _make_sc_gather(rows):
    info = plsc.get_sparse_core_info()
    nc, ns = info.num_cores, info.num_subcores
    nw = nc * ns
    rpw = rows // nw
    stages = rpw // _CH
    mesh = plsc.VectorSubcoreMesh(core_axis_name="c", subcore_axis_name="s")

    @functools.partial(
        pl.kernel,
        mesh=mesh,
        out_type=jax.ShapeDtypeStruct((rows, D), jnp.float32),
        scratch_types=[
            pltpu.VMEM((rpw,), jnp.int32),
            pltpu.VMEM((_CH, D), jnp.float32),
            pltpu.VMEM((_CH, D), jnp.float32),
            pltpu.SemaphoreType.DMA,
            pltpu.SemaphoreType.DMA,
            pltpu.SemaphoreType.DMA,
            pltpu.SemaphoreType.DMA,
        ],
    )
    def sc_gather(x_hbm, idx_hbm, out_hbm, idx_v, buf0, buf1,
                  gs0, gs1, ws0, ws1):
        wid = lax.axis_index("s") * nc + lax.axis_index("c")
        pltpu.sync_copy(idx_hbm.at[pl.ds(wid * rpw, rpw)], idx_v)
        bufs = (buf0, buf1)
        gsems = (gs0, gs1)
        wsems = (ws0, ws1)
        gathers = [None, None]
        writes = [None, None]
        for j in range(stages):
            p = j % 2
            if j >= 2:
                writes[p].wait()
            gathers[p] = pltpu.make_async_copy(
                x_hbm.at[idx_v.at[pl.ds(j * _CH, _CH)]], bufs[p], gsems[p])
            gathers[p].start()
            if j >= 1:
                q = 1 - p
                gathers[q].wait()
                writes[q] = pltpu.make_async_copy(
                    bufs[q], out_hbm.at[pl.ds((wid * stages + j - 1) * _CH,
                                              _CH)], wsems[q])
                writes[q].start()
        pl_ = (stages - 1) % 2
        gathers[pl_].wait()
        writes[pl_] = pltpu.make_async_copy(
            bufs[pl_], out_hbm.at[pl.ds((wid * stages + stages - 1) * _CH,
                                        _CH)], wsems[pl_])
        writes[pl_].start()
        writes[pl_].wait()
        if stages >= 2:
            writes[1 - pl_].wait()

    return sc_gather


def _tc_body(blk0, x_ref, agg_ref, idxf_ref, dw_ref, db_ref, uw_ref, ub_ref,
             gw_ref, gb_ref, *rest):
    out_ref = rest[-1]
    xb = x_ref[...]
    ab = agg_ref[...]
    h = jnp.maximum(
        jnp.dot(ab, dw_ref[...], preferred_element_type=jnp.float32)
        + db_ref[...], 0.0)
    z = jnp.dot(h, uw_ref[...], preferred_element_type=jnp.float32) + ub_ref[...]
    garg = (jnp.dot(xb, gw_ref[:D, :], preferred_element_type=jnp.float32)
            + jnp.dot(ab, gw_ref[D:, :], preferred_element_type=jnp.float32)
            + gb_ref[...])
    g = jax.nn.sigmoid(garg)
    r0 = (blk0 + pl.program_id(0)) * _BT
    rows = r0 + lax.broadcasted_iota(jnp.int32, (_BT, 1), 0)
    t = rows % T
    nbr = jnp.reshape(idxf_ref[...], (_BT, 1)) - (rows // T) * T
    mask = ((t >= 1) & (nbr != 0)).astype(jnp.float32)
    out_ref[...] = xb + (g * z) * mask


def _tc_dense_chunk(blk0, nblk, x2, agg_c, idxf2, dw, db, uw, ub, gw, gb,
                    acc=None):
    in_specs = [
        pl.BlockSpec((_BT, D), lambda i: (blk0 + i, 0)),
        pl.BlockSpec((_BT, D), lambda i: (i, 0)),
        pl.BlockSpec((1, _BT), lambda i: (0, blk0 + i)),
        pl.BlockSpec((D, BOT), lambda i: (0, 0)),
        pl.BlockSpec((1, BOT), lambda i: (0, 0)),
        pl.BlockSpec((BOT, D), lambda i: (0, 0)),
        pl.BlockSpec((1, D), lambda i: (0, 0)),
        pl.BlockSpec((2 * D, D), lambda i: (0, 0)),
        pl.BlockSpec((1, D), lambda i: (0, 0)),
    ]
    args = [x2, agg_c, idxf2, dw, db, uw, ub, gw, gb]
    aliases = {}
    if acc is not None:
        in_specs.append(pl.BlockSpec((8, 128), lambda i: (0, 0)))
        args.append(acc)
        aliases = {9: 0}
    return pl.pallas_call(
        functools.partial(_tc_body, blk0),
        grid=(nblk,),
        in_specs=in_specs,
        out_specs=pl.BlockSpec((_BT, D), lambda i: (blk0 + i, 0)),
        out_shape=jax.ShapeDtypeStruct((N, D), jnp.float32),
        input_output_aliases=aliases,
    )(*args)


def kernel(x, neighbor_idx, down_w, down_b, up_w, up_b, gate_w, gate_b):
    idx = neighbor_idx[:, 0]
    idxf = ((jnp.arange(B, dtype=jnp.int32) * T)[:, None]
            + idx[None, :]).reshape(N)
    x2 = x.reshape(N, D)
    offs = [0]
    for nb in CHUNK_BLOCKS:
        offs.append(offs[-1] + nb)
    aggs = [
        _make_sc_gather(nb * _BT)(
            x2,
            lax.slice(idxf, (offs[c] * _BT,), (offs[c + 1] * _BT,)))
        for c, nb in enumerate(CHUNK_BLOCKS)
    ]
    idxf2 = idxf.reshape(1, N)
    db2, ub2, gb2 = (down_b.reshape(1, BOT), up_b.reshape(1, D),
                     gate_b.reshape(1, D))
    acc = None
    for c, nb in enumerate(CHUNK_BLOCKS):
        acc = _tc_dense_chunk(offs[c], nb, x2, aggs[c], idxf2, down_w, db2,
                              up_w, ub2, gate_w, gb2, acc)
    return acc.reshape(B, T, D)
